# baseline (speedup 1.0000x reference)
import jax
import jax.numpy as jnp
from jax.experimental import pallas as pl
from jax.experimental.pallas import tpu as pltpu

_BLK_L = 256


def _add_kernel(feat_ref, pos_ref, out_ref):
    out_ref[...] = feat_ref[...] + pos_ref[...]


def kernel(feat, pos_table):
    B, L, D = feat.shape
    blk = _BLK_L
    grid = (pl.cdiv(L, blk),)
    return pl.pallas_call(
        _add_kernel,
        grid=grid,
        in_specs=[
            pl.BlockSpec((B, blk, D), lambda l: (0, l, 0)),
            pl.BlockSpec((blk, D), lambda l: (l, 0)),
        ],
        out_specs=pl.BlockSpec((B, blk, D), lambda l: (0, l, 0)),
        out_shape=jax.ShapeDtypeStruct((B, L, D), feat.dtype),
        compiler_params=pltpu.CompilerParams(
            dimension_semantics=("parallel",),
        ),
    )(feat, pos_table)


# speedup vs baseline: 1.0161x; 1.0161x over previous
import jax
import jax.numpy as jnp
from jax.experimental import pallas as pl
from jax.experimental.pallas import tpu as pltpu

_BLK_L = 384


def _add_kernel(feat_ref, pos_ref, out_ref):
    out_ref[...] = feat_ref[...] + pos_ref[...]


def kernel(feat, pos_table):
    B, L, D = feat.shape
    blk = _BLK_L
    grid = (pl.cdiv(L, blk),)
    return pl.pallas_call(
        _add_kernel,
        grid=grid,
        in_specs=[
            pl.BlockSpec((B, blk, D), lambda l: (0, l, 0)),
            pl.BlockSpec((blk, D), lambda l: (l, 0)),
        ],
        out_specs=pl.BlockSpec((B, blk, D), lambda l: (0, l, 0)),
        out_shape=jax.ShapeDtypeStruct((B, L, D), feat.dtype),
        compiler_params=pltpu.CompilerParams(
            dimension_semantics=("parallel",),
        ),
    )(feat, pos_table)
